# baseline (speedup 1.0000x reference)
import functools

import jax
import jax.numpy as jnp
from jax import lax
from jax.experimental import pallas as pl
from jax.experimental.pallas import tpu as pltpu
from jax.experimental.pallas import tpu_sc as plsc

_ROWS = 128
_COLS = 32768
_K = 16384
_ROW_BLOCK = 32

_NC = 2
_NS = 16
_NW = _NC * _NS
_RPW = _ROWS // _NW
_NB1 = 1024
_NB2 = 64
_U = 16
_CAP1 = 4096
_CAP2 = 512


def _keys_body(z_ref, eps_ref, keys_ref):
    eps = jnp.clip(eps_ref[...], 1e-6, 1.0 - 1e-6)
    s = z_ref[...] - jnp.log(-jnp.log(eps))
    b = lax.bitcast_convert_type(s, jnp.int32)
    keys_ref[...] = b ^ ((b >> 31) & jnp.int32(0x7FFFFFFF))


def _tc_keys(z, eps):
    spec = pl.BlockSpec((_ROW_BLOCK, _COLS), lambda i: (i, 0))
    return pl.pallas_call(
        _keys_body,
        grid=(_ROWS // _ROW_BLOCK,),
        in_specs=[spec, spec],
        out_specs=spec,
        out_shape=jax.ShapeDtypeStruct((_ROWS, _COLS), jnp.int32),
    )(z, eps)


_mesh = plsc.VectorSubcoreMesh(core_axis_name="c", subcore_axis_name="s")


@functools.partial(
    pl.kernel,
    out_type=jax.ShapeDtypeStruct((_ROWS, _COLS), jnp.float32),
    mesh=_mesh,
    scratch_types=[
        pltpu.VMEM((_COLS,), jnp.int32),
        pltpu.VMEM((_COLS,), jnp.int32),
        pltpu.VMEM((_COLS,), jnp.float32),
        pltpu.VMEM((_NB1,), jnp.int32),
        pltpu.VMEM((_NB2,), jnp.int32),
        pltpu.VMEM((_CAP1,), jnp.int32),
        pltpu.VMEM((_CAP2,), jnp.int32),
        pltpu.SemaphoreType.DMA,
        pltpu.SemaphoreType.DMA,
    ],
    compiler_params=pltpu.CompilerParams(needs_layout_passes=False),
)
def _sc_select(keys_hbm, out_hbm, kv0, kv1, out_v, h1_v, h2_v, s1_v, s2_v,
               sem_in, sem_out):
    wid = lax.axis_index("s") * _NC + lax.axis_index("c")
    zeros16 = jnp.zeros((16,), jnp.int32)
    ones16 = jnp.ones((16,), jnp.int32)
    lanes = lax.iota(jnp.int32, 16)

    def scan_hist(h_ref, nb, kk):
        def sbody(i, carry):
            ge_acc, ab_acc, tot = carry
            c = (nb // 16 - 1) - i
            t16 = h_ref[pl.ds(c * 16, 16)]
            t_rev = lax.rev(t16, (0,))
            s_rev = plsc.cumsum(t_rev) + tot
            ge = s_rev >= kk
            ge_acc = ge_acc + ge.astype(jnp.int32)
            ab_acc = ab_acc + jnp.where(ge, 0, t_rev)
            tot = tot + jnp.sum(t16)
            return ge_acc, ab_acc, tot
        zv = jnp.zeros((16,), jnp.int32)
        ge_acc, ab_acc, _ = lax.fori_loop(
            0, nb // 16, sbody, (zv, zv, jnp.int32(0)))
        return jnp.sum(ge_acc) - 1, jnp.sum(ab_acc)

    def row_body(r, keys_v, out_ready):
        @plsc.parallel_loop(0, _NB1 // 16, unroll=4)
        def _pz(i):
            h1_v[pl.ds(i * 16, 16)] = zeros16
        for c in range(_NB2 // 16):
            h2_v[pl.ds(c * 16, 16)] = zeros16

        @plsc.parallel_loop(0, _COLS // 16, unroll=_U)
        def _p1(i):
            v = keys_v[pl.ds(i * 16, 16)]
            plsc.addupdate_scatter(h1_v, [(v >> 22) + 512], ones16)
        b1, ca1 = scan_hist(h1_v, _NB1, _K)
        k2 = _K - ca1

        @plsc.parallel_loop(0, _COLS // 16, unroll=_U,
                            carry=jnp.zeros((16,), jnp.int32))
        def _p2(i, off_vec):
            v = keys_v[pl.ds(i * 16, 16)]
            pred = ((v >> 22) + 512) == b1
            plsc.addupdate_scatter(h2_v, [(v >> 16) & 0x3F], ones16, mask=pred)
            pos = off_vec + plsc.cumsum(pred.astype(jnp.int32)) - 1
            pos = jnp.minimum(pos, _CAP1 - 1)
            plsc.store_scatter(s1_v, [pos], v, mask=pred)
            return off_vec + plsc.all_reduce_population_count(pred)
        n1 = jnp.minimum(jnp.max(_p2), _CAP1)
        b2, ca2 = scan_hist(h2_v, _NB2, k2)
        k3 = k2 - ca2
        t_hi = ((b1 - 512) << 6) | b2

        def pc(ci, off_vec):
            v = s1_v[pl.ds(ci * 16, 16)]
            pred = ((v >> 16) == t_hi) & ((ci * 16 + lanes) < n1)
            pos = off_vec + plsc.cumsum(pred.astype(jnp.int32)) - 1
            pos = jnp.minimum(pos, _CAP2 - 1)
            plsc.store_scatter(s2_v, [pos], v, mask=pred)
            return off_vec + plsc.all_reduce_population_count(pred)
        n2 = jnp.minimum(
            jnp.max(lax.fori_loop(0, (n1 + 15) // 16, pc,
                                  jnp.zeros((16,), jnp.int32))),
            _CAP2)
        nch2 = (n2 + 15) // 16

        def sb(i, tlo):
            cand_lo = tlo | (jnp.int32(1) << (15 - i))
            cand = (t_hi << 16) | cand_lo

            def cb(ci, acc):
                v = s2_v[pl.ds(ci * 16, 16)]
                valid = (ci * 16 + lanes) < n2
                return acc + jnp.where(valid & (v >= cand), 1, 0)
            cnt = jnp.sum(lax.fori_loop(0, nch2, cb, jnp.zeros((16,), jnp.int32)))
            return jnp.where(cnt >= k3, cand_lo, tlo)
        tlo = lax.fori_loop(0, 16, sb, jnp.int32(0))
        t = (t_hi << 16) | tlo

        if out_ready is not None:
            out_ready.wait()

        @plsc.parallel_loop(0, _COLS // 16, unroll=_U)
        def _pm(i):
            sl = pl.ds(i * 16, 16)
            out_v[sl] = jnp.where(keys_v[sl] >= t, 1.0, 0.0)
        return pltpu.async_copy(out_v, out_hbm.at[wid * _RPW + r], sem_out)

    kbufs = (kv0, kv1)
    inflight = pltpu.async_copy(keys_hbm.at[wid * _RPW], kbufs[0], sem_in)
    out_ready = None
    for r in range(_RPW):
        inflight.wait()
        if r + 1 < _RPW:
            nxt = pltpu.async_copy(
                keys_hbm.at[wid * _RPW + r + 1], kbufs[(r + 1) % 2], sem_in)
        out_ready = row_body(r, kbufs[r % 2], out_ready)
        if r + 1 < _RPW:
            inflight = nxt
    out_ready.wait()


@jax.jit
def kernel(step, z_loga, eps):
    del step
    keys = _tc_keys(z_loga, eps)
    return _sc_select(keys)


# speedup vs baseline: 1.0100x; 1.0100x over previous
import functools

import jax
import jax.numpy as jnp
from jax import lax
from jax.experimental import pallas as pl
from jax.experimental.pallas import tpu as pltpu
from jax.experimental.pallas import tpu_sc as plsc

_ROWS = 128
_COLS = 32768
_K = 16384
_ROW_BLOCK = 32

_NC = 2
_NS = 16
_NW = _NC * _NS
_RPW = _ROWS // _NW
_NB1 = 1024
_NB2 = 64
_U = 8
_CAP1 = 4096
_CAP2 = 512


def _keys_body(z_ref, eps_ref, keys_ref):
    eps = jnp.clip(eps_ref[...], 1e-6, 1.0 - 1e-6)
    s = z_ref[...] - jnp.log(-jnp.log(eps))
    b = lax.bitcast_convert_type(s, jnp.int32)
    keys_ref[...] = b ^ ((b >> 31) & jnp.int32(0x7FFFFFFF))


def _tc_keys(z, eps):
    spec = pl.BlockSpec((_ROW_BLOCK, _COLS), lambda i: (i, 0))
    return pl.pallas_call(
        _keys_body,
        grid=(_ROWS // _ROW_BLOCK,),
        in_specs=[spec, spec],
        out_specs=spec,
        out_shape=jax.ShapeDtypeStruct((_ROWS, _COLS), jnp.int32),
    )(z, eps)


_mesh = plsc.VectorSubcoreMesh(core_axis_name="c", subcore_axis_name="s")


@functools.partial(
    pl.kernel,
    out_type=jax.ShapeDtypeStruct((_ROWS, _COLS), jnp.float32),
    mesh=_mesh,
    scratch_types=[
        pltpu.VMEM((_COLS,), jnp.int32),
        pltpu.VMEM((_COLS,), jnp.int32),
        pltpu.VMEM((_COLS,), jnp.float32),
        pltpu.VMEM((_NB1,), jnp.int32),
        pltpu.VMEM((_NB2,), jnp.int32),
        pltpu.VMEM((_CAP1,), jnp.int32),
        pltpu.VMEM((_CAP2,), jnp.int32),
        pltpu.SemaphoreType.DMA,
        pltpu.SemaphoreType.DMA,
    ],
    compiler_params=pltpu.CompilerParams(needs_layout_passes=False),
)
def _sc_select(keys_hbm, out_hbm, kv0, kv1, out_v, h1_v, h2_v, s1_v, s2_v,
               sem_in, sem_out):
    wid = lax.axis_index("s") * _NC + lax.axis_index("c")
    zeros16 = jnp.zeros((16,), jnp.int32)
    ones16 = jnp.ones((16,), jnp.int32)
    lanes = lax.iota(jnp.int32, 16)

    def scan_hist(h_ref, nb, kk):
        def sbody(i, carry):
            ge_acc, ab_acc, tot = carry
            c = (nb // 16 - 1) - i
            t16 = h_ref[pl.ds(c * 16, 16)]
            t_rev = lax.rev(t16, (0,))
            s_rev = plsc.cumsum(t_rev) + tot
            ge = s_rev >= kk
            ge_acc = ge_acc + ge.astype(jnp.int32)
            ab_acc = ab_acc + jnp.where(ge, 0, t_rev)
            tot = tot + jnp.sum(t16)
            return ge_acc, ab_acc, tot
        zv = jnp.zeros((16,), jnp.int32)
        ge_acc, ab_acc, _ = lax.fori_loop(
            0, nb // 16, sbody, (zv, zv, jnp.int32(0)))
        return jnp.sum(ge_acc) - 1, jnp.sum(ab_acc)

    def row_body(r, keys_v, out_ready):
        @plsc.parallel_loop(0, _NB1 // 16, unroll=4)
        def _pz(i):
            h1_v[pl.ds(i * 16, 16)] = zeros16
        for c in range(_NB2 // 16):
            h2_v[pl.ds(c * 16, 16)] = zeros16

        @plsc.parallel_loop(0, _COLS // 16, unroll=_U)
        def _p1(i):
            v = keys_v[pl.ds(i * 16, 16)]
            plsc.addupdate_scatter(h1_v, [(v >> 22) + 512], ones16)
        b1, ca1 = scan_hist(h1_v, _NB1, _K)
        k2 = _K - ca1

        @plsc.parallel_loop(0, _COLS // 16, unroll=_U,
                            carry=jnp.zeros((16,), jnp.int32))
        def _p2(i, off_vec):
            v = keys_v[pl.ds(i * 16, 16)]
            pred = ((v >> 22) + 512) == b1
            plsc.addupdate_scatter(h2_v, [(v >> 16) & 0x3F], ones16, mask=pred)
            pos = off_vec + plsc.cumsum(pred.astype(jnp.int32)) - 1
            pos = jnp.minimum(pos, _CAP1 - 1)
            plsc.store_scatter(s1_v, [pos], v, mask=pred)
            return off_vec + plsc.all_reduce_population_count(pred)
        n1 = jnp.minimum(jnp.max(_p2), _CAP1)
        b2, ca2 = scan_hist(h2_v, _NB2, k2)
        k3 = k2 - ca2
        t_hi = ((b1 - 512) << 6) | b2

        def pc(ci, off_vec):
            v = s1_v[pl.ds(ci * 16, 16)]
            pred = ((v >> 16) == t_hi) & ((ci * 16 + lanes) < n1)
            pos = off_vec + plsc.cumsum(pred.astype(jnp.int32)) - 1
            pos = jnp.minimum(pos, _CAP2 - 1)
            plsc.store_scatter(s2_v, [pos], v, mask=pred)
            return off_vec + plsc.all_reduce_population_count(pred)
        n2 = jnp.minimum(
            jnp.max(lax.fori_loop(0, (n1 + 15) // 16, pc,
                                  jnp.zeros((16,), jnp.int32))),
            _CAP2)
        nch2 = (n2 + 15) // 16

        def sb(i, tlo):
            cand_lo = tlo | (jnp.int32(1) << (15 - i))
            cand = (t_hi << 16) | cand_lo

            def cb(ci, acc):
                v = s2_v[pl.ds(ci * 16, 16)]
                valid = (ci * 16 + lanes) < n2
                return acc + jnp.where(valid & (v >= cand), 1, 0)
            cnt = jnp.sum(lax.fori_loop(0, nch2, cb, jnp.zeros((16,), jnp.int32)))
            return jnp.where(cnt >= k3, cand_lo, tlo)
        tlo = lax.fori_loop(0, 16, sb, jnp.int32(0))
        t = (t_hi << 16) | tlo

        if out_ready is not None:
            out_ready.wait()

        @plsc.parallel_loop(0, _COLS // 16, unroll=_U)
        def _pm(i):
            sl = pl.ds(i * 16, 16)
            out_v[sl] = jnp.where(keys_v[sl] >= t, 1.0, 0.0)
        return pltpu.async_copy(out_v, out_hbm.at[wid * _RPW + r], sem_out)

    kbufs = (kv0, kv1)
    inflight = pltpu.async_copy(keys_hbm.at[wid * _RPW], kbufs[0], sem_in)
    out_ready = None
    for r in range(_RPW):
        inflight.wait()
        if r + 1 < _RPW:
            nxt = pltpu.async_copy(
                keys_hbm.at[wid * _RPW + r + 1], kbufs[(r + 1) % 2], sem_in)
        out_ready = row_body(r, kbufs[r % 2], out_ready)
        if r + 1 < _RPW:
            inflight = nxt
    out_ready.wait()


@jax.jit
def kernel(step, z_loga, eps):
    del step
    keys = _tc_keys(z_loga, eps)
    return _sc_select(keys)
